# baseline (speedup 1.0000x reference)
import functools

import jax
import jax.numpy as jnp
from jax import lax
from jax.experimental import pallas as pl
from jax.experimental.pallas import tpu as pltpu
from jax.experimental.pallas import tpu_sc as plsc

VOCAB_ROWS = 100002
DIM = 64
B = 16384
L = 50

NUM_CORES = 2
NUM_SUBCORES = 16
NW = NUM_CORES * NUM_SUBCORES
BBLK = 256
NBLK = B // BBLK
BLK_PER_W = NBLK // NW
TILE_ELEMS = 8 * 128
BLK_WORDS = BBLK * DIM
NGRP = BLK_WORDS // 16


def _make_kernel():
  mesh = plsc.VectorSubcoreMesh(core_axis_name="c", subcore_axis_name="s")

  @functools.partial(
      pl.kernel,
      mesh=mesh,
      compiler_params=pltpu.CompilerParams(use_tc_tiling_on_sc=False,
                                           needs_layout_passes=False),
      out_type=jax.ShapeDtypeStruct((L, DIM // 8, B // 128, 8, 128),
                                    jnp.float32),
      scratch_types=[
          pltpu.VMEM((L, BBLK), jnp.int32),
          pltpu.VMEM((BBLK, DIM), jnp.float32),
          pltpu.VMEM((DIM // 8, BBLK // 128, 8, 128), jnp.float32),
          pltpu.SemaphoreType.DMA,
          pltpu.SemaphoreType.DMA,
          pltpu.SemaphoreType.DMA,
      ],
  )
  def gather_kernel(idxt_hbm, table_hbm, out_hbm, idx_v, g_v, t_v,
                    isem, gsem, wsem):
    wid = lax.axis_index("s") * NUM_CORES + lax.axis_index("c")
    lane = lax.iota(jnp.int32, 16)

    for blk in range(BLK_PER_W):
      j = wid * BLK_PER_W + blk
      b0 = j * BBLK
      pltpu.async_copy(idxt_hbm.at[:, pl.ds(b0, BBLK)], idx_v, isem)
      pltpu.make_async_copy(idxt_hbm.at[:, pl.ds(b0, BBLK)], idx_v,
                            isem).wait()

      def chunk(l, carry):
        pltpu.async_copy(table_hbm.at[idx_v.at[l]], g_v, gsem)
        pltpu.make_async_copy(table_hbm.at[idx_v.at[l]], g_v, gsem).wait()

        def grp(o, carry2):
          o0 = o * 16
          dt = o0 >> 11
          bt = (o0 >> 10) & 1
          ds = (o0 >> 7) & 7
          bs0 = o0 & 127
          d = dt * 8 + ds
          r0 = bt * 128 + bs0
          vals = plsc.load_gather(g_v, [lane + r0, jnp.full((16,), d,
                                                            jnp.int32)])
          t_v[dt, bt, ds, pl.ds(bs0, 16)] = vals
          return carry2

        lax.fori_loop(0, NGRP, grp, 0, unroll=4)

        for dt in range(DIM // 8):
          pltpu.async_copy(t_v.at[dt],
                           out_hbm.at[l, dt, pl.ds(j * 2, 2)], wsem)
        for dt in range(DIM // 8):
          pltpu.make_async_copy(t_v.at[dt],
                                out_hbm.at[l, dt, pl.ds(j * 2, 2)],
                                wsem).wait()
        return carry

      lax.fori_loop(0, L, chunk, 0)

  return gather_kernel


_gather = _make_kernel()


@jax.jit
def kernel(idx_input, table):
  idx_t = idx_input.T.astype(jnp.int32)
  out5 = _gather(idx_t, table)
  return out5.transpose(2, 4, 0, 1, 3).reshape(B, L, DIM)


# speedup vs baseline: 1.1551x; 1.1551x over previous
import functools

import jax
import jax.numpy as jnp
from jax import lax
from jax.experimental import pallas as pl
from jax.experimental.pallas import tpu as pltpu
from jax.experimental.pallas import tpu_sc as plsc

VOCAB_ROWS = 100002
DIM = 64
B = 16384
L = 50

NUM_CORES = 2
NUM_SUBCORES = 16
NW = NUM_CORES * NUM_SUBCORES
WIN = B // NW
CBLK = 256
NH = WIN // CBLK
NCHUNK = L * NH
NGRP = CBLK // 16


def _make_kernel():
  mesh = plsc.VectorSubcoreMesh(core_axis_name="c", subcore_axis_name="s")

  @functools.partial(
      pl.kernel,
      mesh=mesh,
      compiler_params=pltpu.CompilerParams(use_tc_tiling_on_sc=False,
                                           needs_layout_passes=False),
      out_type=jax.ShapeDtypeStruct((L, DIM // 8, B // 128, 8, 128),
                                    jnp.float32),
      scratch_types=[
          pltpu.VMEM((L, WIN), jnp.int32),
          pltpu.VMEM((CBLK, DIM), jnp.float32),
          pltpu.VMEM((CBLK, DIM), jnp.float32),
          pltpu.VMEM((DIM // 8, CBLK // 128, 8, 128), jnp.float32),
          pltpu.VMEM((DIM // 8, CBLK // 128, 8, 128), jnp.float32),
          pltpu.SemaphoreType.DMA,
          pltpu.SemaphoreType.DMA,
          pltpu.SemaphoreType.DMA,
          pltpu.SemaphoreType.DMA,
          pltpu.SemaphoreType.DMA,
      ],
  )
  def gather_kernel(idxt_hbm, table_hbm, out_hbm, idx_v, g0, g1, t0, t1,
                    isem, gsem0, gsem1, wsem0, wsem1):
    wid = lax.axis_index("s") * NUM_CORES + lax.axis_index("c")
    b0 = wid * WIN
    lane = lax.iota(jnp.int32, 16)
    rvec = [lane + 16 * k for k in range(NGRP)]

    pltpu.async_copy(idxt_hbm.at[:, pl.ds(b0, WIN)], idx_v, isem)
    pltpu.make_async_copy(idxt_hbm.at[:, pl.ds(b0, WIN)], idx_v, isem).wait()

    def start_gather(c, g, sem):
      pltpu.async_copy(
          table_hbm.at[idx_v.at[c // 2, pl.ds((c % 2) * CBLK, CBLK)]], g, sem)

    def wait_gather(c, g, sem):
      pltpu.make_async_copy(
          table_hbm.at[idx_v.at[c // 2, pl.ds((c % 2) * CBLK, CBLK)]], g,
          sem).wait()

    def retile(g, t):
      def body(d, carry):
        dt = d >> 3
        ds = d & 7
        col = jnp.full((16,), d, jnp.int32)
        for k in range(NGRP):
          vals = plsc.load_gather(g, [rvec[k], col])
          t[dt, k >> 3, ds, pl.ds((k & 7) * 16, 16)] = vals
        return carry

      lax.fori_loop(0, DIM, body, 0)

    def tile_col(c):
      return wid * (2 * NH) + (c % 2) * 2

    def start_write(c, t, sem):
      pltpu.async_copy(t, out_hbm.at[c // 2, :, pl.ds(tile_col(c), 2)], sem)

    def wait_write(c, t, sem):
      pltpu.make_async_copy(
          t, out_hbm.at[c // 2, :, pl.ds(tile_col(c), 2)], sem).wait()

    start_gather(0, g0, gsem0)
    start_gather(1, g1, gsem1)

    wait_gather(0, g0, gsem0)
    retile(g0, t0)
    start_gather(2, g0, gsem0)
    start_write(0, t0, wsem0)
    wait_gather(1, g1, gsem1)
    retile(g1, t1)
    start_gather(3, g1, gsem1)
    start_write(1, t1, wsem1)

    def body(i, carry):
      c0 = 2 * i
      c1 = c0 + 1
      wait_gather(c0, g0, gsem0)
      wait_write(c0 - 2, t0, wsem0)
      retile(g0, t0)
      start_gather(c0 + 2, g0, gsem0)
      start_write(c0, t0, wsem0)
      wait_gather(c1, g1, gsem1)
      wait_write(c1 - 2, t1, wsem1)
      retile(g1, t1)
      start_gather(c1 + 2, g1, gsem1)
      start_write(c1, t1, wsem1)
      return carry

    lax.fori_loop(1, NCHUNK // 2 - 1, body, 0)

    cl0 = NCHUNK - 2
    cl1 = NCHUNK - 1
    wait_gather(cl0, g0, gsem0)
    wait_write(cl0 - 2, t0, wsem0)
    retile(g0, t0)
    start_write(cl0, t0, wsem0)
    wait_gather(cl1, g1, gsem1)
    wait_write(cl1 - 2, t1, wsem1)
    retile(g1, t1)
    start_write(cl1, t1, wsem1)
    wait_write(cl0, t0, wsem0)
    wait_write(cl1, t1, wsem1)

  return gather_kernel


_gather = _make_kernel()


@jax.jit
def kernel(idx_input, table):
  idx_t = idx_input.T.astype(jnp.int32)
  out5 = _gather(idx_t, table)
  return out5.transpose(2, 4, 0, 1, 3).reshape(B, L, DIM)


# speedup vs baseline: 2.0477x; 1.7728x over previous
import functools

import jax
import jax.numpy as jnp
from jax import lax
from jax.experimental import pallas as pl
from jax.experimental.pallas import tpu as pltpu
from jax.experimental.pallas import tpu_sc as plsc

VOCAB_ROWS = 100002
DIM = 64
B = 16384
L = 50

NUM_CORES = 2
NUM_SUBCORES = 16
NW = NUM_CORES * NUM_SUBCORES
WIN = B // NW
CBLK = 256
NH = WIN // CBLK
NCHUNK = L * NH
NGRP = CBLK // 16


def _make_kernel():
  mesh = plsc.VectorSubcoreMesh(core_axis_name="c", subcore_axis_name="s")

  @functools.partial(
      pl.kernel,
      mesh=mesh,
      compiler_params=pltpu.CompilerParams(use_tc_tiling_on_sc=False,
                                           needs_layout_passes=False),
      out_type=jax.ShapeDtypeStruct((L, DIM // 8, B // 128, 8, 128),
                                    jnp.float32),
      scratch_types=[
          pltpu.VMEM((L, WIN), jnp.int32),
          pltpu.VMEM((CBLK, DIM), jnp.float32),
          pltpu.VMEM((CBLK, DIM), jnp.float32),
          pltpu.VMEM((DIM // 8, CBLK // 128, 8, 128), jnp.float32),
          pltpu.VMEM((DIM // 8, CBLK // 128, 8, 128), jnp.float32),
          pltpu.SemaphoreType.DMA,
          pltpu.SemaphoreType.DMA,
          pltpu.SemaphoreType.DMA,
          pltpu.SemaphoreType.DMA,
          pltpu.SemaphoreType.DMA,
      ],
  )
  def gather_kernel(idxt_hbm, table_hbm, out_hbm, idx_v, g0, g1, t0, t1,
                    isem, gsem0, gsem1, wsem0, wsem1):
    wid = lax.axis_index("s") * NUM_CORES + lax.axis_index("c")
    b0 = wid * WIN
    lane = lax.iota(jnp.int32, 16)
    rvec = [lane + 16 * k for k in range(NGRP)]

    pltpu.async_copy(idxt_hbm.at[:, pl.ds(b0, WIN)], idx_v, isem)
    pltpu.make_async_copy(idxt_hbm.at[:, pl.ds(b0, WIN)], idx_v, isem).wait()

    def start_gather(c, g, sem):
      pltpu.async_copy(
          table_hbm.at[idx_v.at[c // 2, pl.ds((c % 2) * CBLK, CBLK)]], g, sem)

    def wait_gather(c, g, sem):
      pltpu.make_async_copy(
          table_hbm.at[idx_v.at[c // 2, pl.ds((c % 2) * CBLK, CBLK)]], g,
          sem).wait()

    def retile(g, t):
      @plsc.parallel_loop(0, DIM, unroll=4)
      def body(d):
        dt = d >> 3
        ds = d & 7
        col = jnp.full((16,), d, jnp.int32)
        for k in range(NGRP):
          vals = plsc.load_gather(g, [rvec[k], col])
          t[dt, k >> 3, ds, pl.ds((k & 7) * 16, 16)] = vals

    def tile_col(c):
      return wid * (2 * NH) + (c % 2) * 2

    def start_write(c, t, sem):
      pltpu.async_copy(t, out_hbm.at[c // 2, :, pl.ds(tile_col(c), 2)], sem)

    def wait_write(c, t, sem):
      pltpu.make_async_copy(
          t, out_hbm.at[c // 2, :, pl.ds(tile_col(c), 2)], sem).wait()

    start_gather(0, g0, gsem0)
    start_gather(1, g1, gsem1)

    wait_gather(0, g0, gsem0)
    retile(g0, t0)
    start_gather(2, g0, gsem0)
    start_write(0, t0, wsem0)
    wait_gather(1, g1, gsem1)
    retile(g1, t1)
    start_gather(3, g1, gsem1)
    start_write(1, t1, wsem1)

    def body(i, carry):
      c0 = 2 * i
      c1 = c0 + 1
      wait_gather(c0, g0, gsem0)
      wait_write(c0 - 2, t0, wsem0)
      retile(g0, t0)
      start_gather(c0 + 2, g0, gsem0)
      start_write(c0, t0, wsem0)
      wait_gather(c1, g1, gsem1)
      wait_write(c1 - 2, t1, wsem1)
      retile(g1, t1)
      start_gather(c1 + 2, g1, gsem1)
      start_write(c1, t1, wsem1)
      return carry

    lax.fori_loop(1, NCHUNK // 2 - 1, body, 0)

    cl0 = NCHUNK - 2
    cl1 = NCHUNK - 1
    wait_gather(cl0, g0, gsem0)
    wait_write(cl0 - 2, t0, wsem0)
    retile(g0, t0)
    start_write(cl0, t0, wsem0)
    wait_gather(cl1, g1, gsem1)
    wait_write(cl1 - 2, t1, wsem1)
    retile(g1, t1)
    start_write(cl1, t1, wsem1)
    wait_write(cl0, t0, wsem0)
    wait_write(cl1, t1, wsem1)

  return gather_kernel


_gather = _make_kernel()


@jax.jit
def kernel(idx_input, table):
  idx_t = idx_input.T.astype(jnp.int32)
  out5 = _gather(idx_t, table)
  return out5.transpose(2, 4, 0, 1, 3).reshape(B, L, DIM)
